# baseline (speedup 1.0000x reference)
import jax
import jax.numpy as jnp
from jax.experimental import pallas as pl
from jax.experimental.pallas import tpu as pltpu

_IN_C = 64
_OUT_C = 64
_KERN = 3
_B = 4
_V = 4096
_FB = _IN_C * _B
_OB = _OUT_C * _B
_BS = 512
_NS = _V // _BS
_BM = 512
_NP = _V // _BM
_G = _NP + (_NS + 1) + _NP


def _gconv_body(inp_ref, L_ref, Wcat_ref, bias_ref, y_ref,
                Lb_ref, x0b_ref, v_ref):
    g = pl.program_id(0)

    @pl.when(g < _NP)
    def _pack():
        rows = pl.ds(g * _BM, _BM)
        for bb in range(_B):
            x0b_ref[rows, bb * _IN_C:(bb + 1) * _IN_C] = (
                inp_ref[bb, :, :].astype(jnp.bfloat16))

    @pl.when((g >= _NP) & (g < _NP + _NS + 1))
    def _stream():
        s = g - _NP

        @pl.when(s < _NS)
        def _cast():
            Lb_ref[pl.ds(s * _BS, _BS), :] = L_ref[...].astype(jnp.bfloat16)

        @pl.when(s >= 1)
        def _vdot():
            rows = pl.ds((s - 1) * _BS, _BS)
            x1b = jnp.dot(
                Lb_ref[rows, :], x0b_ref[...],
                preferred_element_type=jnp.float32).astype(jnp.bfloat16)
            v = jnp.dot(x0b_ref[rows, :], Wcat_ref[_FB:2 * _FB, :],
                        preferred_element_type=jnp.float32)
            v += 2.0 * jnp.dot(x1b, Wcat_ref[2 * _FB:3 * _FB, :],
                               preferred_element_type=jnp.float32)
            v_ref[rows, :] = v.astype(jnp.bfloat16)

    @pl.when(g >= _NP + _NS + 1)
    def _pass2():
        rows = pl.ds((g - (_NP + _NS + 1)) * _BM, _BM)
        acc = jnp.dot(Lb_ref[rows, :], v_ref[...],
                      preferred_element_type=jnp.float32)
        acc += jnp.dot(x0b_ref[rows, :], Wcat_ref[0:_FB, :],
                       preferred_element_type=jnp.float32)
        acc += bias_ref[...]
        for bb in range(_B):
            y_ref[bb, :, :] = acc[:, bb * _OUT_C:(bb + 1) * _OUT_C]


def kernel(inp, L, W, b):
    Bn, Vn, Fin = inp.shape

    Wr = W.reshape(_OUT_C, Fin, _KERN)
    core = jnp.transpose(Wr, (2, 1, 0))
    eye = jnp.eye(Bn, dtype=W.dtype)
    Wk = jnp.einsum('kfo,ab->kafbo', core, eye).reshape(
        _KERN, Bn * Fin, Bn * _OUT_C)
    Wcat = jnp.concatenate([Wk[0] - Wk[2], Wk[1], Wk[2]],
                           axis=0).astype(jnp.bfloat16)
    bias_big = jnp.tile(b, Bn).reshape(1, Bn * _OUT_C)

    y = pl.pallas_call(
        _gconv_body,
        grid=(_G,),
        in_specs=[
            pl.BlockSpec((Bn, _BM, Fin),
                         lambda g: (0, jnp.clip(g, 0, _NP - 1), 0)),
            pl.BlockSpec((_BS, Vn),
                         lambda g: (jnp.clip(g - _NP, 0, _NS - 1), 0)),
            pl.BlockSpec((_KERN * _FB, _OB), lambda g: (0, 0)),
            pl.BlockSpec((1, _OB), lambda g: (0, 0)),
        ],
        out_specs=pl.BlockSpec(
            (Bn, _BM, _OUT_C),
            lambda g: (0, jnp.clip(g - (_NP + _NS + 1), 0, _NP - 1), 0)),
        out_shape=jax.ShapeDtypeStruct((Bn, Vn, _OUT_C), jnp.float32),
        scratch_shapes=[
            pltpu.VMEM((Vn, Vn), jnp.bfloat16),
            pltpu.VMEM((Vn, _FB), jnp.bfloat16),
            pltpu.VMEM((Vn, _FB), jnp.bfloat16),
        ],
        compiler_params=pltpu.CompilerParams(
            dimension_semantics=("arbitrary",)),
    )(inp, L, Wcat, bias_big)
    return y


# speedup vs baseline: 1.0107x; 1.0107x over previous
import jax
import jax.numpy as jnp
from jax.experimental import pallas as pl
from jax.experimental.pallas import tpu as pltpu

_IN_C = 64
_OUT_C = 64
_KERN = 3
_B = 4
_V = 4096
_FB = _IN_C * _B
_OB = _OUT_C * _B
_BS = 512
_NS = _V // _BS
_BM = 512
_NP = _V // _BM
_G = _NP + _NS + _NP


def _gconv_body(inp_ref, L_ref, Wbig_ref, bias_ref, y_ref,
                Lb_ref, x0b_ref, x1_ref):
    g = pl.program_id(0)

    @pl.when(g < _NP)
    def _pack():
        rows = pl.ds(g * _BM, _BM)
        for bb in range(_B):
            x0b_ref[rows, bb * _IN_C:(bb + 1) * _IN_C] = (
                inp_ref[bb, :, :].astype(jnp.bfloat16))

        @pl.when(g == _NP - 1)
        def _cast0():
            Lb_ref[0:_BS, :] = L_ref[...].astype(jnp.bfloat16)

    @pl.when((g >= _NP) & (g < _NP + _NS))
    def _stream():
        s = g - _NP

        @pl.when(s < _NS - 1)
        def _cast_next():
            Lb_ref[pl.ds((s + 1) * _BS, _BS), :] = (
                L_ref[...].astype(jnp.bfloat16))

        rows = pl.ds(s * _BS, _BS)
        x1_ref[rows, :] = jnp.dot(
            Lb_ref[rows, :], x0b_ref[...],
            preferred_element_type=jnp.float32).astype(jnp.bfloat16)

    @pl.when(g >= _NP + _NS)
    def _pass2():
        rows = pl.ds((g - (_NP + _NS)) * _BM, _BM)
        x0_blk = x0b_ref[rows, :]
        x2_blk = 2.0 * jnp.dot(
            Lb_ref[rows, :], x1_ref[...],
            preferred_element_type=jnp.float32) - x0_blk.astype(jnp.float32)
        acc = jnp.dot(x0_blk, Wbig_ref[0:_FB, :],
                      preferred_element_type=jnp.float32)
        acc += jnp.dot(x1_ref[rows, :], Wbig_ref[_FB:2 * _FB, :],
                       preferred_element_type=jnp.float32)
        acc += jnp.dot(x2_blk.astype(jnp.bfloat16),
                       Wbig_ref[2 * _FB:3 * _FB, :],
                       preferred_element_type=jnp.float32)
        acc += bias_ref[...]
        for bb in range(_B):
            y_ref[bb, :, :] = acc[:, bb * _OUT_C:(bb + 1) * _OUT_C]


def kernel(inp, L, W, b):
    Bn, Vn, Fin = inp.shape

    Wr = W.reshape(_OUT_C, Fin, _KERN)
    core = jnp.transpose(Wr, (2, 1, 0))
    eye = jnp.eye(Bn, dtype=W.dtype)
    Wbig = jnp.einsum('kfo,ab->kafbo', core, eye).reshape(
        _KERN * Bn * Fin, Bn * _OUT_C).astype(jnp.bfloat16)
    bias_big = jnp.tile(b, Bn).reshape(1, Bn * _OUT_C)

    y = pl.pallas_call(
        _gconv_body,
        grid=(_G,),
        in_specs=[
            pl.BlockSpec((Bn, _BM, Fin),
                         lambda g: (0, jnp.clip(g, 0, _NP - 1), 0)),
            pl.BlockSpec((_BS, Vn),
                         lambda g: (jnp.clip(g - (_NP - 1), 0, _NS - 1), 0)),
            pl.BlockSpec((_KERN * _FB, _OB), lambda g: (0, 0)),
            pl.BlockSpec((1, _OB), lambda g: (0, 0)),
        ],
        out_specs=pl.BlockSpec(
            (Bn, _BM, _OUT_C),
            lambda g: (0, jnp.clip(g - (_NP + _NS), 0, _NP - 1), 0)),
        out_shape=jax.ShapeDtypeStruct((Bn, Vn, _OUT_C), jnp.float32),
        scratch_shapes=[
            pltpu.VMEM((Vn, Vn), jnp.bfloat16),
            pltpu.VMEM((Vn, _FB), jnp.bfloat16),
            pltpu.VMEM((Vn, _FB), jnp.bfloat16),
        ],
        compiler_params=pltpu.CompilerParams(
            dimension_semantics=("arbitrary",)),
    )(inp, L, Wbig, bias_big)
    return y
